# baseline (speedup 1.0000x reference)
import functools

import jax
import jax.numpy as jnp
from jax import lax
from jax.experimental import pallas as pl
from jax.experimental.pallas import tpu as pltpu
from jax.experimental.pallas import tpu_sc as plsc


def _make_depad(n_nodes: int, n_out: int, channels: int):
    info = plsc.get_sparse_core_info()
    nc, ns = info.num_cores, info.num_subcores
    nw = nc * ns
    assert n_out % nw == 0
    bpw = n_out // nw
    chunk = 128
    assert bpw % chunk == 0
    n_chunks = bpw // chunk

    mesh = plsc.VectorSubcoreMesh(core_axis_name="c", subcore_axis_name="s")

    @functools.partial(
        pl.kernel,
        mesh=mesh,
        compiler_params=pltpu.CompilerParams(use_tc_tiling_on_sc=False),
        out_type=jax.ShapeDtypeStruct((n_out, channels), jnp.float32),
        scratch_types=[
            pltpu.VMEM((bpw,), jnp.int32),
            pltpu.VMEM((chunk, channels), jnp.float32),
            pltpu.SemaphoreType.DMA,
        ],
    )
    def depad(data_hbm, idx_hbm, out_hbm, idx_v, rows_v, sem):
        wid = lax.axis_index("s") * nc + lax.axis_index("c")
        base = wid * bpw
        pltpu.sync_copy(idx_hbm.at[pl.ds(base, bpw)], idx_v)

        def body(j, carry):
            off = j * chunk
            pltpu.async_copy(
                data_hbm.at[idx_v.at[pl.ds(off, chunk)]], rows_v, sem
            ).wait()
            pltpu.sync_copy(rows_v, out_hbm.at[pl.ds(base + off, chunk)])
            return carry

        lax.fori_loop(0, n_chunks, body, 0)

    return depad


def kernel(data_in, nempty_idx):
    n_nodes, channels = data_in.shape
    n_out = nempty_idx.shape[0]
    depad = _make_depad(n_nodes, n_out, channels)
    return depad(data_in, nempty_idx)


# speedup vs baseline: 1.0804x; 1.0804x over previous
import functools

import jax
import jax.numpy as jnp
from jax import lax
from jax.experimental import pallas as pl
from jax.experimental.pallas import tpu as pltpu
from jax.experimental.pallas import tpu_sc as plsc


def _make_depad(n_nodes: int, n_out: int, channels: int):
    info = plsc.get_sparse_core_info()
    nc, ns = info.num_cores, info.num_subcores
    nw = nc * ns
    assert n_out % nw == 0
    bpw = n_out // nw
    chunk = 256
    nbuf = 4
    dist = 2
    assert bpw % chunk == 0
    n_chunks = bpw // chunk
    assert n_chunks > nbuf >= 2 * dist

    mesh = plsc.VectorSubcoreMesh(core_axis_name="c", subcore_axis_name="s")

    @functools.partial(
        pl.kernel,
        mesh=mesh,
        compiler_params=pltpu.CompilerParams(use_tc_tiling_on_sc=False),
        out_type=jax.ShapeDtypeStruct((n_out, channels), jnp.float32),
        scratch_types=[
            pltpu.VMEM((bpw,), jnp.int32),
            pltpu.VMEM((nbuf, chunk, channels), jnp.float32),
            [pltpu.SemaphoreType.DMA] * nbuf,
            [pltpu.SemaphoreType.DMA] * nbuf,
        ],
    )
    def depad(data_hbm, idx_hbm, out_hbm, idx_v, bufs, gsems, wsems):
        wid = lax.axis_index("s") * nc + lax.axis_index("c")
        base = wid * bpw
        pltpu.sync_copy(idx_hbm.at[pl.ds(base, bpw)], idx_v)

        def gather(j):
            b = j % nbuf
            return pltpu.make_async_copy(
                data_hbm.at[idx_v.at[pl.ds(j * chunk, chunk)]],
                bufs.at[b],
                gsems[b],
            )

        def writeback(j):
            b = j % nbuf
            return pltpu.make_async_copy(
                bufs.at[b],
                out_hbm.at[pl.ds(base + j * chunk, chunk)],
                wsems[b],
            )

        for j in range(dist):
            gather(j).start()
        for j in range(n_chunks):
            gather(j).wait()
            writeback(j).start()
            if j + dist < n_chunks:
                if j - dist >= 0:
                    writeback(j - dist).wait()
                gather(j + dist).start()
        for j in range(n_chunks - dist, n_chunks):
            writeback(j).wait()

    return depad


def kernel(data_in, nempty_idx):
    n_nodes, channels = data_in.shape
    n_out = nempty_idx.shape[0]
    depad = _make_depad(n_nodes, n_out, channels)
    return depad(data_in, nempty_idx)
